# baseline (speedup 1.0000x reference)
import functools

import jax
import jax.numpy as jnp
from jax import lax
from jax.experimental import pallas as pl
from jax.experimental.pallas import tpu as pltpu
from jax.experimental.pallas import tpu_sc as plsc

B = 16384
DIM = 64
MARGIN = 1.0

NC, NS, L = 2, 16, 16
NW = NC * NS
ROWS_PER_W = B // NW
CHUNK = 128
NCHUNK = ROWS_PER_W // CHUNK


def _sc_sqdist(ent_idx, rel_idx, ent_emb, rel_emb):
  mesh = plsc.VectorSubcoreMesh(core_axis_name="c", subcore_axis_name="s")

  @functools.partial(
      pl.kernel,
      mesh=mesh,
      compiler_params=pltpu.CompilerParams(
          needs_layout_passes=False, use_tc_tiling_on_sc=False),
      out_type=jax.ShapeDtypeStruct((2, B), jnp.float32),
      scratch_types=[
          pltpu.VMEM((CHUNK,), jnp.int32),
          pltpu.VMEM((CHUNK,), jnp.int32),
          pltpu.VMEM((CHUNK,), jnp.int32),
          pltpu.VMEM((CHUNK, DIM), jnp.float32),
          pltpu.VMEM((CHUNK, DIM), jnp.float32),
          pltpu.VMEM((CHUNK, DIM), jnp.float32),
          pltpu.VMEM((L * L,), jnp.float32),
          pltpu.VMEM((CHUNK,), jnp.float32),
          pltpu.SemaphoreType.DMA,
      ],
  )
  def k(ent_idx_hbm, rel_idx_hbm, ent_hbm, rel_hbm, out_hbm,
        idxh_v, idxr_v, idxt_v, h_v, r_v, t_v, acc16_v, ss_v, sem):
    wid = lax.axis_index("s") * NC + lax.axis_index("c")
    base = wid * ROWS_PER_W
    iota = jnp.arange(L, dtype=jnp.int32)
    for side in range(2):
      for ck in range(NCHUNK):
        off = base + ck * CHUNK
        pltpu.sync_copy(ent_idx_hbm.at[2 * side, pl.ds(off, CHUNK)], idxh_v)
        pltpu.sync_copy(rel_idx_hbm.at[side, pl.ds(off, CHUNK)], idxr_v)
        pltpu.sync_copy(ent_idx_hbm.at[2 * side + 1, pl.ds(off, CHUNK)], idxt_v)
        ch = pltpu.async_copy(ent_hbm.at[idxh_v], h_v, sem)
        cr = pltpu.async_copy(rel_hbm.at[idxr_v], r_v, sem)
        ct = pltpu.async_copy(ent_hbm.at[idxt_v], t_v, sem)
        ch.wait()
        cr.wait()
        ct.wait()

        def group(g, _):
          rbase = pl.multiple_of(g * L, L)
          for r in range(L):
            i = rbase + r
            acc = jnp.zeros((L,), jnp.float32)
            for j in range(DIM // L):
              d = (h_v[i, pl.ds(j * L, L)] + r_v[i, pl.ds(j * L, L)]
                   - t_v[i, pl.ds(j * L, L)])
              acc = acc + d * d
            acc16_v[pl.ds(r * L, L)] = acc
          tot = jnp.zeros((L,), jnp.float32)
          for j in range(L):
            tot = tot + plsc.load_gather(acc16_v, [iota * L + j])
          ss_v[pl.ds(rbase, L)] = tot
          return 0

        lax.fori_loop(0, CHUNK // L, group, 0)
        pltpu.sync_copy(ss_v, out_hbm.at[side, pl.ds(off, CHUNK)])

  return k(ent_idx, rel_idx, ent_emb, rel_emb)


def _tc_loss(pos_ss, neg_ss):
  def body(p_ref, n_ref, o_ref):
    p = jnp.sqrt(p_ref[...])
    n = jnp.sqrt(n_ref[...])
    v = jnp.maximum(p - n + MARGIN, 0.0)
    o_ref[...] = (jnp.sum(v) * (1.0 / B)).reshape(1, 1)

  return pl.pallas_call(
      body,
      out_shape=jax.ShapeDtypeStruct((1, 1), jnp.float32),
  )(pos_ss, neg_ss)


def kernel(pos_triples, neg_triples, ent_emb, rel_emb):
  pt = pos_triples.astype(jnp.int32)
  nt = neg_triples.astype(jnp.int32)
  ent_idx = jnp.stack([pt[:, 0], pt[:, 2], nt[:, 0], nt[:, 2]])
  rel_idx = jnp.stack([pt[:, 1], nt[:, 1]])
  ss = _sc_sqdist(ent_idx, rel_idx, ent_emb, rel_emb)
  loss = _tc_loss(ss[0].reshape(128, 128), ss[1].reshape(128, 128))
  return loss[0, 0]
